# Initial kernel scaffold; baseline (speedup 1.0000x reference)
import jax
import jax.numpy as jnp
from jax.experimental import pallas as pl


def kernel(x, w_gate, w_noise, W1, b1, W2, b2):
    raise NotImplementedError("write your pallas kernel here")


# speedup vs baseline: 1.1898x; 1.1898x over previous
import functools

import jax
import jax.numpy as jnp
from jax.experimental import pallas as pl
from jax.experimental.pallas import tpu as pltpu

NUM_EXPERTS = 8
TOP_K = 2
D_IN = 1024
D_HID = 4096
D_OUT = 1024
N_TOK = 2048

BLK = 256
GMAX = 24
PPAD = GMAX * BLK
NH = 4
HB = D_HID // NH


def _gating(x, w_gate):
    logits = x @ w_gate
    p = jax.nn.softmax(logits, axis=1)
    lane = jnp.arange(NUM_EXPERTS)[None, :]
    m1 = jnp.max(p, 1, keepdims=True)
    i1 = jnp.min(jnp.where(p == m1, lane, NUM_EXPERTS), 1, keepdims=True)
    p2 = jnp.where(lane == i1, -jnp.inf, p)
    m2 = jnp.max(p2, 1, keepdims=True)
    i2 = jnp.min(jnp.where(p2 == m2, lane, NUM_EXPERTS), 1, keepdims=True)
    denom = m1 + m2 + 1e-6
    g1, g2 = m1 / denom, m2 / denom
    oh1 = (lane == i1).astype(jnp.float32)
    oh2 = (lane == i2).astype(jnp.float32)
    importance = (oh1 * g1 + oh2 * g2).sum(0)
    load = (oh1 + oh2).sum(0)

    def cv2(v):
        mu = v.mean()
        var = ((v - mu) ** 2).sum() / (NUM_EXPERTS - 1)
        return var / (mu * mu + 1e-10)

    loss = 1e-2 * (cv2(importance) + cv2(load))
    return i1[:, 0], i2[:, 0], g1[:, 0], g2[:, 0], loss


def _route(i1, i2, g1, g2):
    pe = jnp.stack([i1, i2], 1).reshape(-1)
    pg = jnp.stack([g1, g2], 1).reshape(-1)
    pt = jnp.repeat(jnp.arange(N_TOK), TOP_K)
    onehot = (pe[:, None] == jnp.arange(NUM_EXPERTS)[None, :]).astype(jnp.int32)
    counts = onehot.sum(0)
    rank = jnp.cumsum(onehot, 0) - 1
    rank = jnp.take_along_axis(rank, pe[:, None], 1)[:, 0]
    nb = (counts + BLK - 1) // BLK
    cum_nb = jnp.cumsum(nb)
    bstart = cum_nb - nb
    dest = bstart[pe] * BLK + rank
    sorted_tid = jnp.zeros(PPAD, jnp.int32).at[dest].set(pt.astype(jnp.int32))
    sorted_gate = jnp.zeros(PPAD, jnp.float32).at[dest].set(pg)
    pos = dest.reshape(N_TOK, TOP_K)
    g_used = cum_nb[NUM_EXPERTS - 1]
    bidx = jnp.arange(GMAX)
    be = jnp.searchsorted(cum_nb, bidx, side="right")
    be_last = jnp.searchsorted(cum_nb, g_used - 1, side="right")
    bv = (bidx < g_used).astype(jnp.int32)
    be = jnp.where(bv == 1, be, be_last).astype(jnp.int32)
    return sorted_tid, sorted_gate, be, bv, pos


def _ffn_body(be_ref, bv_ref, gate_ref, xb_ref, w1_ref, b1_ref, w2_ref,
              b2_ref, out_ref, acc_ref):
    g = pl.program_id(0)
    h = pl.program_id(1)

    @pl.when(bv_ref[g] == 1)
    def _():
        xb = xb_ref[...]
        hb = jnp.dot(xb, w1_ref[0], preferred_element_type=jnp.float32)
        hb = jnp.maximum(hb + b1_ref[0], 0.0)
        contrib = jnp.dot(hb, w2_ref[0], preferred_element_type=jnp.float32)

        @pl.when(h == 0)
        def _():
            acc_ref[...] = contrib

        @pl.when(h > 0)
        def _():
            acc_ref[...] += contrib

        @pl.when(h == NH - 1)
        def _():
            logits = acc_ref[...] + b2_ref[0]
            m = jnp.max(logits, axis=1, keepdims=True)
            ex = jnp.exp(logits - m)
            o = ex / jnp.sum(ex, axis=1, keepdims=True)
            out_ref[...] = o * gate_ref[0, 0][:, None]


@functools.partial(jax.jit, static_argnames=())
def _ffn(x_sorted, gate3d, W1, b1, W2, b2, be, bv):
    grid_spec = pltpu.PrefetchScalarGridSpec(
        num_scalar_prefetch=2,
        grid=(GMAX, NH),
        in_specs=[
            pl.BlockSpec((1, 1, BLK), lambda g, h, be, bv: (g, 0, 0)),
            pl.BlockSpec((BLK, D_IN), lambda g, h, be, bv: (g, 0)),
            pl.BlockSpec((1, D_IN, HB), lambda g, h, be, bv: (be[g], 0, h)),
            pl.BlockSpec((1, 1, HB), lambda g, h, be, bv: (be[g] * NH + h, 0, 0)),
            pl.BlockSpec((1, HB, D_OUT), lambda g, h, be, bv: (be[g], h, 0)),
            pl.BlockSpec((1, 1, D_OUT), lambda g, h, be, bv: (be[g], 0, 0)),
        ],
        out_specs=pl.BlockSpec((BLK, D_OUT), lambda g, h, be, bv: (g, 0)),
        scratch_shapes=[pltpu.VMEM((BLK, D_OUT), jnp.float32)],
    )
    return pl.pallas_call(
        _ffn_body,
        grid_spec=grid_spec,
        out_shape=jax.ShapeDtypeStruct((PPAD, D_OUT), jnp.float32),
        compiler_params=pltpu.CompilerParams(
            dimension_semantics=("arbitrary", "arbitrary")),
    )(be, bv, gate3d, x_sorted, W1,
      b1.reshape(NUM_EXPERTS * NH, 1, HB), W2,
      b2.reshape(NUM_EXPERTS, 1, D_OUT))


def kernel(x, w_gate, w_noise, W1, b1, W2, b2):
    i1, i2, g1, g2, loss = _gating(x, w_gate)
    sorted_tid, sorted_gate, be, bv, pos = _route(i1, i2, g1, g2)
    x_sorted = x[sorted_tid]
    gate3d = sorted_gate.reshape(GMAX, 1, BLK)
    o_sorted = _ffn(x_sorted, gate3d, W1, b1, W2, b2, be, bv)
    y = o_sorted[pos[:, 0]] + o_sorted[pos[:, 1]]
    return y, loss
